# baseline (speedup 1.0000x reference)
import functools

import jax
import jax.numpy as jnp
from jax import lax
from jax.experimental import pallas as pl
from jax.experimental.pallas import tpu as pltpu
from jax.experimental.pallas import tpu_sc as plsc

N_VOCAB = 100000
D = 64
BATCH = 1024
SEQ = 200
OUT_SEQ = SEQ + 1
NC = 2
NS = 16
NW = NC * NS
BC = 128
UNITS = SEQ * (BATCH // BC)
UPW = UNITS // NW
GRP = 5
GPW = UPW // GRP
SP = (OUT_SEQ + 1) // 2
MID_ROWS = SP * BATCH

_mesh = plsc.VectorSubcoreMesh(core_axis_name="c", subcore_axis_name="s")


@functools.partial(
    pl.kernel,
    mesh=_mesh,
    out_type=jax.ShapeDtypeStruct((MID_ROWS, 2 * D), jnp.float32),
    scratch_types=[
        pltpu.VMEM((UPW, BC), jnp.int32),
        pltpu.VMEM((GRP * BC, D), jnp.float32),
        pltpu.VMEM((GRP * BC, D), jnp.float32),
        pltpu.VMEM((1, 2 * NS), jnp.int32),
        pltpu.VMEM((BATCH // NW, D), jnp.float32),
        pltpu.SemaphoreType.DMA,
        pltpu.SemaphoreType.DMA,
    ],
    compiler_params=pltpu.CompilerParams(use_tc_tiling_on_sc=False),
)
def _embed_gather(ids_hbm, table_hbm, out_hbm, idx_v, buf0, buf1, zidx, bosb, sem0, sem1):
    wid = lax.axis_index("s") * NC + lax.axis_index("c")
    u_base = wid * UPW

    pltpu.sync_copy(ids_hbm.at[pl.ds(u_base, UPW)], idx_v)

    zidx[0, pl.ds(0, 16)] = jnp.zeros((16,), jnp.int32)
    zidx[0, pl.ds(16, 16)] = jnp.zeros((16,), jnp.int32)
    pltpu.async_copy(table_hbm.at[zidx.at[0]], bosb, sem0).wait()
    pltpu.sync_copy(
        bosb, out_hbm.at[pl.ds(wid * (BATCH // NW), BATCH // NW), pl.ds(0, D)]
    )

    def issue_group(g, buf, sem):
        for j in range(GRP):
            u = u_base + g * GRP + j
            dst = buf.at[pl.ds(j * BC, BC)]
            pltpu.async_copy(table_hbm.at[idx_v.at[g * GRP + j]], dst, sem)

    def drain_group(buf, sem):
        pltpu.make_async_copy(
            table_hbm.at[pl.ds(0, GRP * BC)], buf, sem
        ).wait()

    def write_group(g, buf):
        for j in range(GRP):
            u = u_base + g * GRP + j
            s = 1 + u // 8
            row0 = (s // 2) * BATCH + (u % 8) * BC
            dst = out_hbm.at[pl.ds(row0, BC), pl.ds((s % 2) * D, D)]
            pltpu.sync_copy(buf.at[pl.ds(j * BC, BC)], dst)

    issue_group(0, buf0, sem0)

    def body(i, carry):
        g = 2 * i
        issue_group(g + 1, buf1, sem1)
        drain_group(buf0, sem0)
        write_group(g, buf0)

        @pl.when(i < GPW // 2 - 1)
        def _():
            issue_group(g + 2, buf0, sem0)

        drain_group(buf1, sem1)
        write_group(g + 1, buf1)
        return carry

    lax.fori_loop(0, GPW // 2, body, 0)


def _table_body(xa_ref, xb_ref, y_ref):
    y_ref[...] = jnp.concatenate(
        [xa_ref[...].transpose(), xb_ref[...].transpose()], axis=1
    )


PAIR = 196 * 256

_tc_table = pl.pallas_call(
    _table_body,
    grid=(196,),
    in_specs=[
        pl.BlockSpec((D, 256), lambda i: (0, i)),
        pl.BlockSpec((D, 256), lambda i: (0, i + 196)),
    ],
    out_specs=pl.BlockSpec((256, 2 * D), lambda i: (i, 0)),
    out_shape=jax.ShapeDtypeStruct((PAIR, 2 * D), jnp.float32),
)


def _transpose_body(x_ref, y_ref):
    x = x_ref[...]
    y_ref[0] = x[:, :D].transpose()
    y_ref[1] = x[:, D:].transpose()


_tc_transpose = pl.pallas_call(
    _transpose_body,
    grid=(SP,),
    in_specs=[pl.BlockSpec((BATCH, 2 * D), lambda i: (i, 0))],
    out_specs=pl.BlockSpec((2, D, BATCH), lambda i: (i, 0, 0)),
    out_shape=jax.ShapeDtypeStruct((OUT_SEQ, D, BATCH), jnp.float32),
)


def kernel(input_ids, embedding_weight):
    ids_v = jnp.where(input_ids < PAIR, 2 * input_ids, 2 * input_ids - (2 * PAIR - 1))
    ids2 = ids_v.T.reshape(UNITS, BC)
    xt = embedding_weight.T
    wt_lin = _tc_table(xt, xt).reshape(2 * PAIR, D)
    mid = _embed_gather(ids2, wt_lin)
    y = _tc_transpose(mid)
    return jnp.transpose(y, (2, 0, 1))


# speedup vs baseline: 1.4832x; 1.4832x over previous
import functools

import jax
import jax.numpy as jnp
from jax import lax
from jax.experimental import pallas as pl
from jax.experimental.pallas import tpu as pltpu
from jax.experimental.pallas import tpu_sc as plsc

N_VOCAB = 100000
D = 64
BATCH = 1024
SEQ = 200
OUT_SEQ = SEQ + 1
NC = 2
NS = 16
NW = NC * NS
BC = 128
UNITS = SEQ * (BATCH // BC)
UPW = UNITS // NW
GRP = 5
GPW = UPW // GRP
SP = (OUT_SEQ + 1) // 2
MID_ROWS = SP * BATCH

_mesh = plsc.VectorSubcoreMesh(core_axis_name="c", subcore_axis_name="s")


@functools.partial(
    pl.kernel,
    mesh=_mesh,
    out_type=jax.ShapeDtypeStruct((MID_ROWS, 2 * D), jnp.float32),
    scratch_types=[
        pltpu.VMEM((UPW, BC), jnp.int32),
        pltpu.VMEM((GRP * BC, D), jnp.float32),
        pltpu.VMEM((GRP * BC, D), jnp.float32),
        pltpu.VMEM((1, 2 * NS), jnp.int32),
        pltpu.VMEM((BATCH // NW, D), jnp.float32),
        pltpu.SemaphoreType.DMA,
        pltpu.SemaphoreType.DMA,
    ],
    compiler_params=pltpu.CompilerParams(use_tc_tiling_on_sc=False),
)
def _embed_gather(ids_hbm, table_hbm, out_hbm, idx_v, buf0, buf1, zidx, bosb, sem0, sem1):
    wid = lax.axis_index("s") * NC + lax.axis_index("c")
    u_base = wid * UPW

    pltpu.sync_copy(ids_hbm.at[pl.ds(u_base, UPW)], idx_v)

    zidx[0, pl.ds(0, 16)] = jnp.zeros((16,), jnp.int32)
    zidx[0, pl.ds(16, 16)] = jnp.zeros((16,), jnp.int32)
    pltpu.async_copy(table_hbm.at[zidx.at[0]], bosb, sem0).wait()
    pltpu.sync_copy(
        bosb, out_hbm.at[pl.ds(wid * (BATCH // NW), BATCH // NW), pl.ds(0, D)]
    )

    def issue_group(g, buf, sem):
        for j in range(GRP):
            u = u_base + g * GRP + j
            dst = buf.at[pl.ds(j * BC, BC)]
            pltpu.async_copy(table_hbm.at[idx_v.at[g * GRP + j]], dst, sem)

    def drain_group(buf, sem):
        pltpu.make_async_copy(
            table_hbm.at[pl.ds(0, GRP * BC)], buf, sem
        ).wait()

    def write_group(g, buf):
        for j in range(GRP):
            u = u_base + g * GRP + j
            s = 1 + u // 8
            row0 = (s // 2) * BATCH + (u % 8) * BC
            dst = out_hbm.at[pl.ds(row0, BC), pl.ds((s % 2) * D, D)]
            pltpu.sync_copy(buf.at[pl.ds(j * BC, BC)], dst)

    issue_group(0, buf0, sem0)

    def body(i, carry):
        g = 2 * i
        issue_group(g + 1, buf1, sem1)
        drain_group(buf0, sem0)
        write_group(g, buf0)

        @pl.when(i < GPW // 2 - 1)
        def _():
            issue_group(g + 2, buf0, sem0)

        drain_group(buf1, sem1)
        write_group(g + 1, buf1)
        return carry

    lax.fori_loop(0, GPW // 2, body, 0)


def _table_body(xa_ref, xb_ref, y_ref):
    y_ref[...] = jnp.concatenate([xa_ref[...], xb_ref[...]], axis=0).transpose()


PAIR = 49 * 1024

_tc_table = pl.pallas_call(
    _table_body,
    grid=(49,),
    in_specs=[
        pl.BlockSpec((D, 1024), lambda i: (0, i)),
        pl.BlockSpec((D, 1024), lambda i: (0, i + 49)),
    ],
    out_specs=pl.BlockSpec((1024, 2 * D), lambda i: (i, 0)),
    out_shape=jax.ShapeDtypeStruct((PAIR, 2 * D), jnp.float32),
)


def _transpose_body(x_ref, y_ref):
    xt = x_ref[...].transpose()
    y_ref[0] = xt[:D]
    y_ref[1] = xt[D:]


_tc_transpose = pl.pallas_call(
    _transpose_body,
    grid=(SP,),
    in_specs=[pl.BlockSpec((BATCH, 2 * D), lambda i: (i, 0))],
    out_specs=pl.BlockSpec((2, D, BATCH), lambda i: (i, 0, 0)),
    out_shape=jax.ShapeDtypeStruct((OUT_SEQ, D, BATCH), jnp.float32),
)


def kernel(input_ids, embedding_weight):
    ids_v = jnp.where(input_ids < PAIR, 2 * input_ids, 2 * input_ids - (2 * PAIR - 1))
    ids2 = ids_v.T.reshape(UNITS, BC)
    xt = embedding_weight.T
    wt_lin = _tc_table(xt, xt).reshape(2 * PAIR, D)
    mid = _embed_gather(ids2, wt_lin)
    y = _tc_transpose(mid)
    return jnp.transpose(y, (2, 0, 1))
